# Initial kernel scaffold; baseline (speedup 1.0000x reference)
import jax
import jax.numpy as jnp
from jax.experimental import pallas as pl


def kernel(x, edge_index, W1, b1, W2, b2, Wfc, bfc):
    raise NotImplementedError("write your pallas kernel here")


# speedup vs baseline: 51.0298x; 51.0298x over previous
import functools

import jax
import jax.numpy as jnp
from jax import lax
from jax.experimental import pallas as pl
from jax.experimental.pallas import tpu as pltpu
from jax.experimental.pallas import tpu_sc as plsc

_NC = 2
_NS = 16
_LANES = 16
_B = 1024


def _pad_rows(N):
    return ((N + 512 - 1) // 512) * 512


def _fast_rsqrt(x):
    i = lax.bitcast_convert_type(x, jnp.int32)
    i = jnp.int32(0x5F3759DF) - lax.shift_right_logical(i, 1)
    y = lax.bitcast_convert_type(i, jnp.float32)
    xh = x * 0.5
    for _ in range(3):
        y = y * (1.5 - xh * y * y)
    return y


def _make_deg_dinv(N, E):
    Np = ((N + (128 * _NS) - 1) // (128 * _NS)) * (128 * _NS)
    per_tile = Np // _NS
    half = Np // _NC
    per_tile_out = half // _NS
    NB = E // _B
    assert NB * _B == E
    mesh = plsc.VectorSubcoreMesh(core_axis_name="c", subcore_axis_name="s")

    @functools.partial(
        pl.kernel,
        out_type=jax.ShapeDtypeStruct((Np,), jnp.float32),
        mesh=mesh,
        scratch_types=[
            pltpu.VMEM((_B,), jnp.int32),
            pltpu.VMEM((per_tile,), jnp.float32),
            pltpu.VMEM((per_tile,), jnp.float32),
            pltpu.VMEM_SHARED((Np,), jnp.float32),
            pltpu.SemaphoreType.DMA,
        ],
        compiler_params=pltpu.CompilerParams(use_tc_tiling_on_sc=False),
    )
    def k(dst_hbm, out_hbm, dst_v, ones_v, stage_v, acc_sh, sem):
        c = lax.axis_index("c")
        s = lax.axis_index("s")
        one16 = jnp.full((16,), 1.0, jnp.float32)

        def fill_ones(i, _):
            ones_v[pl.ds(i * 16, 16)] = one16
            return 0

        lax.fori_loop(0, per_tile // 16, fill_ones, 0)
        pltpu.sync_copy(ones_v, acc_sh.at[pl.ds(s * per_tile, per_tile)])
        plsc.subcore_barrier()

        lo = s * NB // _NS
        hi = (s + 1) * NB // _NS

        def body(b, _):
            pltpu.sync_copy(dst_hbm.at[pl.ds(b * _B, _B)], dst_v)
            pltpu.async_copy(
                ones_v.at[pl.ds(0, _B)], acc_sh.at[dst_v], sem, add=True
            ).wait()
            return 0

        lax.fori_loop(lo, hi, body, 0)
        plsc.subcore_barrier()

        off = c * half + s * per_tile_out
        pltpu.sync_copy(acc_sh.at[pl.ds(off, per_tile_out)],
                        stage_v.at[pl.ds(0, per_tile_out)])

        def rsq(i, _):
            v = stage_v[pl.ds(i * 16, 16)]
            stage_v[pl.ds(i * 16, 16)] = _fast_rsqrt(v)
            return 0

        lax.fori_loop(0, per_tile_out // 16, rsq, 0)
        pltpu.sync_copy(stage_v.at[pl.ds(0, per_tile_out)],
                        out_hbm.at[pl.ds(off, per_tile_out)])

    return k


def _make_edge_pass(N, E, feature_split):
    NB = E // _B
    assert NB * _B == E
    Nr = _pad_rows(N)
    rows_tile = Nr // _NS
    CH = rows_tile // 8
    nch = 8
    mesh = plsc.VectorSubcoreMesh(core_axis_name="c", subcore_axis_name="s")

    @functools.partial(
        pl.kernel,
        out_type=jax.ShapeDtypeStruct((_NC, Nr, 16), jnp.float32),
        mesh=mesh,
        scratch_types=[
            pltpu.VMEM((_B,), jnp.int32),
            pltpu.VMEM((_B,), jnp.int32),
            pltpu.VMEM((_B, 16), jnp.float32),
            pltpu.VMEM_SHARED((Nr, 16), jnp.float32),
            pltpu.SemaphoreType.DMA,
            pltpu.SemaphoreType.DMA,
        ],
        compiler_params=pltpu.CompilerParams(use_tc_tiling_on_sc=False),
    )
    def k(src_hbm, dst_hbm, tab_hbm, zeros_hbm, out_hbm,
          src_v, dst_v, rows_v, acc_sh, gsem, ssem):
        c = lax.axis_index("c")
        s = lax.axis_index("s")

        stage = rows_v.at[pl.ds(0, CH), :]
        pltpu.sync_copy(zeros_hbm, stage)
        row0 = s * rows_tile
        for j in range(nch):
            pltpu.sync_copy(stage, acc_sh.at[pl.ds(row0 + j * CH, CH), :])
        plsc.subcore_barrier()

        if feature_split:
            lo = s * NB // _NS
            hi = (s + 1) * NB // _NS
        else:
            wid = s * _NC + c
            lo = wid * NB // (_NC * _NS)
            hi = (wid + 1) * NB // (_NC * _NS)

        def body(b, _):
            base = b * _B
            pltpu.sync_copy(src_hbm.at[pl.ds(base, _B)], src_v)
            pltpu.sync_copy(dst_hbm.at[pl.ds(base, _B)], dst_v)
            if feature_split:
                off16 = jnp.full((16,), c * N, jnp.int32)

                def addoff(i, _):
                    src_v[pl.ds(i * 16, 16)] = src_v[pl.ds(i * 16, 16)] + off16
                    return 0

                lax.fori_loop(0, _B // 16, addoff, 0)
            pltpu.async_copy(tab_hbm.at[src_v], rows_v, gsem).wait()
            pltpu.async_copy(rows_v, acc_sh.at[dst_v], ssem, add=True).wait()
            return 0

        lax.fori_loop(lo, hi, body, 0)
        plsc.subcore_barrier()

        for j in range(nch):
            r = row0 + j * CH
            pltpu.sync_copy(acc_sh.at[pl.ds(r, CH), :], stage)
            pltpu.sync_copy(stage, out_hbm.at[c, pl.ds(r, CH), :])

    return k


def _tc_g1(x, W1, dinv_col, N):
    R = 2000
    G = N // R

    def body(x_ref, w_ref, d_ref, g1_ref):
        g = jnp.dot(x_ref[...], w_ref[...], preferred_element_type=jnp.float32)
        g1_ref[...] = g * d_ref[...]

    return pl.pallas_call(
        body,
        grid=(G,),
        in_specs=[
            pl.BlockSpec((R, 4), lambda i: (i, 0)),
            pl.BlockSpec((4, 16), lambda i: (0, 0)),
            pl.BlockSpec((R, 1), lambda i: (i, 0)),
        ],
        out_specs=pl.BlockSpec((R, 16), lambda i: (i, 0)),
        out_shape=jax.ShapeDtypeStruct((N, 16), jnp.float32),
    )(x, W1, dinv_col)


def _tc_mid(acc1, g1, dinv_col, b1_row, W2, N):
    R = 2000
    G = N // R

    def body(a_ref, g1_ref, d_ref, b_ref, w_ref, out_ref):
        s1 = a_ref[0] + a_ref[1] + g1_ref[...]
        h1 = jnp.maximum(s1 * d_ref[...] + b_ref[...], 0.0)
        g2 = jnp.dot(h1, w_ref[...], preferred_element_type=jnp.float32)
        g2 = g2 * d_ref[...]
        out_ref[0] = g2[:, :16]
        out_ref[1] = g2[:, 16:]

    return pl.pallas_call(
        body,
        grid=(G,),
        in_specs=[
            pl.BlockSpec((2, R, 16), lambda i: (0, i, 0)),
            pl.BlockSpec((R, 16), lambda i: (i, 0)),
            pl.BlockSpec((R, 1), lambda i: (i, 0)),
            pl.BlockSpec((1, 16), lambda i: (0, 0)),
            pl.BlockSpec((16, 32), lambda i: (0, 0)),
        ],
        out_specs=pl.BlockSpec((2, R, 16), lambda i: (0, i, 0)),
        out_shape=jax.ShapeDtypeStruct((2, N, 16), jnp.float32),
    )(acc1, g1, dinv_col, b1_row, W2)


def _tc_head(acc2, g2sp, dinv_col, b2_row, Wfc, bfc_row, N):
    R = 2000
    G = N // R

    def body(a_ref, g_ref, d_ref, b2_ref, w_ref, bfc_ref, out_ref):
        s2 = jnp.concatenate([a_ref[0] + g_ref[0], a_ref[1] + g_ref[1]], axis=1)
        h2 = jnp.maximum(s2 * d_ref[...] + b2_ref[...], 0.0)
        out_ref[...] = (
            jnp.dot(h2, w_ref[...], preferred_element_type=jnp.float32)
            + bfc_ref[...]
        )

    return pl.pallas_call(
        body,
        grid=(G,),
        in_specs=[
            pl.BlockSpec((2, R, 16), lambda i: (0, i, 0)),
            pl.BlockSpec((2, R, 16), lambda i: (0, i, 0)),
            pl.BlockSpec((R, 1), lambda i: (i, 0)),
            pl.BlockSpec((1, 32), lambda i: (0, 0)),
            pl.BlockSpec((32, 4), lambda i: (0, 0)),
            pl.BlockSpec((1, 4), lambda i: (0, 0)),
        ],
        out_specs=pl.BlockSpec((R, 4), lambda i: (i, 0)),
        out_shape=jax.ShapeDtypeStruct((N, 4), jnp.float32),
    )(acc2, g2sp, dinv_col, b2_row, Wfc, bfc_row)


def kernel(x, edge_index, W1, b1, W2, b2, Wfc, bfc):
    N = x.shape[0]
    E = edge_index.shape[1]
    src = edge_index[0]
    dst = edge_index[1]

    dinv_p = _make_deg_dinv(N, E)(dst)
    dinv_col = dinv_p[:N].reshape(N, 1)
    zeros_stage = jnp.zeros((_pad_rows(N) // _NS // 8, 16), jnp.float32)

    g1 = _tc_g1(x, W1, dinv_col, N)
    acc1 = _make_edge_pass(N, E, feature_split=False)(src, dst, g1, zeros_stage)

    g2sp = _tc_mid(acc1, g1, dinv_col, b1.reshape(1, 16), W2, N)
    g2flat = g2sp.reshape(2 * N, 16)
    acc2 = _make_edge_pass(N, E, feature_split=True)(src, dst, g2flat,
                                                     zeros_stage)

    return _tc_head(acc2, g2sp, dinv_col, b2.reshape(1, 32), Wfc,
                    bfc.reshape(1, 4), N)
